# baseline (speedup 1.0000x reference)
import functools

import jax
import jax.numpy as jnp
from jax import lax
from jax.experimental import pallas as pl
from jax.experimental.pallas import tpu as pltpu
from jax.experimental.pallas import tpu_sc as plsc

N = 10000
D = 128
K = 16
R2 = 0.25
NEG = -1e30
BIG = 1e30

NP = 10240
BLK_A = 64
BLK_B = 64
E = NP * K
CC = 512
D2 = 2 * D


def _knn_body(lo_ref, nc_ref, x_ref, posr_ref, posc_ref, batr_ref, batc_ref,
              wp1_ref, nbr_ref, val_ref, tab_ref, d_s):
    pid = pl.program_id(0)
    lo = lo_ref[pid]
    nc = nc_ref[pid]

    pr = posr_ref[...]
    br = batr_ref[...]

    tab_ref[:, :D] = x_ref[...]
    tab_ref[:, D:] = jnp.dot(pr, wp1_ref[...],
                             preferred_element_type=jnp.float32)

    iota_c = lax.broadcasted_iota(jnp.int32, (BLK_A, CC), 1)

    def fill(c, _):
        off = (lo + c) * CC
        pcc = posc_ref[:, pl.ds(off, CC)]
        bcc = batc_ref[:, pl.ds(off, CC)]
        d2 = ((pr[:, 0:1] - pcc[0:1, :]) ** 2
              + (pr[:, 1:2] - pcc[1:2, :]) ** 2
              + (pr[:, 2:3] - pcc[2:3, :]) ** 2)
        ok = (br == bcc) & (d2 <= R2)
        d_s[:, pl.ds(off, CC)] = jnp.where(ok, d2, BIG)
        return 0

    lax.fori_loop(0, nc, fill, 0)

    m0 = jnp.full((BLK_A, 1), BIG, jnp.float32)
    i0 = jnp.zeros((BLK_A, 1), jnp.int32)

    for k in range(K):
        def scan(c, carry):
            m, ix = carry
            off = (lo + c) * CC
            dch = d_s[:, pl.ds(off, CC)]
            mc = jnp.min(dch, axis=1, keepdims=True)
            cand = jnp.where(dch == mc, iota_c + off, jnp.int32(NP))
            ic = jnp.min(cand, axis=1, keepdims=True)
            take = mc < m
            return jnp.where(take, mc, m), jnp.where(take, ic, ix)

        m, ix = lax.fori_loop(0, nc, scan, (m0, i0))

        def clear(c, _):
            off = (lo + c) * CC
            d_s[:, pl.ds(off, CC)] = jnp.where(
                iota_c + off == ix, BIG, d_s[:, pl.ds(off, CC)])
            return 0

        lax.fori_loop(0, nc, clear, 0)

        good = m <= R2
        nbr_ref[:, k:k + 1] = jnp.where(good, ix, 0)
        val_ref[:, k:k + 1] = jnp.where(good, 1.0, 0.0)


def _knn(lo, nc, x_p, posr, posc, batr, batc, W_pos1):
    grid = NP // BLK_A
    smem = pl.BlockSpec(memory_space=pltpu.SMEM)
    return pl.pallas_call(
        _knn_body,
        grid=(grid,),
        in_specs=[
            smem, smem,
            pl.BlockSpec((BLK_A, D), lambda i: (i, 0)),
            pl.BlockSpec((BLK_A, 3), lambda i: (i, 0)),
            pl.BlockSpec((3, NP), lambda i: (0, 0)),
            pl.BlockSpec((BLK_A, 1), lambda i: (i, 0)),
            pl.BlockSpec((1, NP), lambda i: (0, 0)),
            pl.BlockSpec((3, D), lambda i: (0, 0)),
        ],
        out_specs=[
            pl.BlockSpec((BLK_A, K), lambda i: (i, 0)),
            pl.BlockSpec((BLK_A, K), lambda i: (i, 0)),
            pl.BlockSpec((BLK_A, D2), lambda i: (i, 0)),
        ],
        out_shape=[
            jax.ShapeDtypeStruct((NP, K), jnp.int32),
            jax.ShapeDtypeStruct((NP, K), jnp.float32),
            jax.ShapeDtypeStruct((NP, D2), jnp.float32),
        ],
        scratch_shapes=[pltpu.VMEM((BLK_A, NP), jnp.float32)],
    )(lo, nc, x_p, posr, posc, batr, batc, W_pos1)


_SC_CHUNK = 128
_SC_NBUF = 3


def _sc_gather(idx_flat, tab):
    info = plsc.get_sparse_core_info()
    nw = info.num_cores * info.num_subcores
    epw = E // nw
    nch = epw // _SC_CHUNK
    mesh = plsc.VectorSubcoreMesh(core_axis_name="c", subcore_axis_name="s")

    @functools.partial(
        pl.kernel,
        out_type=jax.ShapeDtypeStruct((E, D2), jnp.float32),
        mesh=mesh,
        scratch_types=[
            pltpu.VMEM((epw,), jnp.int32),
            pltpu.VMEM((_SC_NBUF, _SC_CHUNK, D2), jnp.float32),
        ] + [pltpu.SemaphoreType.DMA] * (2 * _SC_NBUF),
    )
    def gather(idx_hbm, tab_hbm, tg_out, idx_v, buf, *sems):
        gsem = sems[:_SC_NBUF]
        wsem = sems[_SC_NBUF:]
        wid = lax.axis_index("s") * info.num_cores + lax.axis_index("c")
        w0 = wid * epw
        pltpu.sync_copy(idx_hbm.at[pl.ds(w0, epw)], idx_v)

        g = {}
        wb = {}

        def flush(c):
            g[c].wait()
            b = c % _SC_NBUF
            base = w0 + c * _SC_CHUNK
            wb[c] = pltpu.async_copy(
                buf.at[b], tg_out.at[pl.ds(base, _SC_CHUNK)], wsem[b])

        for c in range(nch):
            b = c % _SC_NBUF
            if c >= _SC_NBUF:
                wb[c - _SC_NBUF].wait()
            isl = idx_v.at[pl.ds(c * _SC_CHUNK, _SC_CHUNK)]
            g[c] = pltpu.async_copy(tab_hbm.at[isl], buf.at[b], gsem[b])
            if c >= 1:
                flush(c - 1)
        flush(nch - 1)
        for c in range(nch - _SC_NBUF, nch):
            wb[c].wait()

    return gather(idx_flat, tab)


def _block_body(tab_ref, tg_ref, val_ref,
                wlin_ref, wsrc_ref, wdst_ref, bp1_ref,
                wp2_ref, bp2_ref, watt_ref, batt_ref, g_ref, b_ref,
                out_ref):
    f32 = jnp.float32
    xg = tg_ref[:, :D]
    a_src = jnp.dot(xg, wsrc_ref[...], preferred_element_type=f32)
    v_e = jnp.dot(xg, wlin_ref[...], preferred_element_type=f32)

    q = tab_ref[:, D:]
    qg = tg_ref[:, D:].reshape(BLK_B, K, D)
    h = jnp.maximum(
        (q[:, None, :] - qg).reshape(BLK_B * K, D) + bp1_ref[...], 0.0)
    delta = jnp.maximum(
        jnp.dot(h, wp2_ref[...], preferred_element_type=f32) + bp2_ref[...],
        0.0)

    a_dst = jnp.dot(tab_ref[:, :D], wdst_ref[...], preferred_element_type=f32)
    ai = (a_dst[:, None, :] - a_src.reshape(BLK_B, K, D)
          + delta.reshape(BLK_B, K, D)).reshape(BLK_B * K, D)
    alpha = jnp.maximum(
        jnp.dot(ai, watt_ref[...], preferred_element_type=f32) + batt_ref[...],
        0.0)

    v3 = val_ref[...][:, :, None] > 0.0
    al3 = jnp.where(v3, alpha.reshape(BLK_B, K, D), NEG)
    mx = jnp.max(al3, axis=1, keepdims=True)
    ex = jnp.exp(al3 - mx)
    sm = ex / jnp.sum(ex, axis=1, keepdims=True)
    sm = jnp.where(v3, sm, 0.0)

    msg = sm * (v_e.reshape(BLK_B, K, D) + delta.reshape(BLK_B, K, D))
    out = jnp.sum(msg, axis=1)

    mu = jnp.mean(out, axis=-1, keepdims=True)
    var = jnp.mean((out - mu) ** 2, axis=-1, keepdims=True)
    y = (out - mu) / jnp.sqrt(var + 1e-5)
    out_ref[...] = y * g_ref[...] + b_ref[...]


def _block(tab, tg, validf, W_lin, W_src, W_dst,
           b_pos1, W_pos2, b_pos2, W_att, b_att, gamma, beta):
    grid = NP // BLK_B
    full = lambda r, c: pl.BlockSpec((r, c), lambda i: (0, 0))
    return pl.pallas_call(
        _block_body,
        grid=(grid,),
        in_specs=[
            pl.BlockSpec((BLK_B, D2), lambda i: (i, 0)),
            pl.BlockSpec((BLK_B * K, D2), lambda i: (i, 0)),
            pl.BlockSpec((BLK_B, K), lambda i: (i, 0)),
            full(D, D), full(D, D), full(D, D),
            full(1, D),
            full(D, D), full(1, D),
            full(D, D), full(1, D),
            full(1, D), full(1, D),
        ],
        out_specs=pl.BlockSpec((BLK_B, D), lambda i: (i, 0)),
        out_shape=jax.ShapeDtypeStruct((NP, D), jnp.float32),
    )(tab, tg, validf, W_lin, W_src, W_dst,
      b_pos1, W_pos2, b_pos2, W_att, b_att, gamma, beta)


def kernel(x, pos, batch, W_lin, W_src, W_dst, W_pos1, b_pos1,
           W_pos2, b_pos2, W_att, b_att, gamma, beta):
    batch = batch.astype(jnp.int32)

    posr = jnp.pad(pos, ((0, NP - N), (0, 0)))
    posc = posr.T
    batr = jnp.pad(batch, (0, NP - N), constant_values=-1).reshape(NP, 1)
    batc = jnp.pad(batch, (0, NP - N), constant_values=-2).reshape(1, NP)
    x_p = jnp.pad(x, ((0, NP - N), (0, 0)))

    grid = NP // BLK_A
    seg_start = jnp.searchsorted(batch, jnp.arange(8), side="left")
    seg_end = jnp.searchsorted(batch, jnp.arange(8), side="right")
    row0 = jnp.arange(grid) * BLK_A
    b0 = jnp.clip(batch[jnp.minimum(row0, N - 1)], 0, 7)
    b1 = jnp.clip(batch[jnp.minimum(row0 + BLK_A - 1, N - 1)], 0, 7)
    lo_col = seg_start[b0]
    hi_col = seg_end[b1]
    lo = (lo_col // CC).astype(jnp.int32)
    nc = ((hi_col + CC - 1) // CC).astype(jnp.int32) - lo
    nc = jnp.maximum(nc, 1)

    nbr, validf, tab = _knn(lo, nc, x_p, posr, posc, batr, batc, W_pos1)

    tg = _sc_gather(nbr.reshape(E), tab)

    y = _block(tab, tg, validf,
               W_lin, W_src, W_dst, b_pos1.reshape(1, D),
               W_pos2, b_pos2.reshape(1, D), W_att, b_att.reshape(1, D),
               gamma.reshape(1, D), beta.reshape(1, D))

    return y[:N], pos, batch
